# Initial kernel scaffold; baseline (speedup 1.0000x reference)
import jax
import jax.numpy as jnp
from jax.experimental import pallas as pl


def kernel(encoded_row, encoded_col, W, robot_lot_idx, robot_lot_step, flow, num_lot_type, num_step):
    raise NotImplementedError("write your pallas kernel here")


# speedup vs baseline: 2.3711x; 2.3711x over previous
import functools

import jax
import jax.numpy as jnp
from jax import lax
from jax.experimental import pallas as pl
from jax.experimental.pallas import tpu as pltpu
from jax.experimental.pallas import tpu_sc as plsc


def _sc_gather(row_flat, col_flat, flow_flat, idx_am, step_am,
               boff_row, boff_flow, boff_col, nlt_vec, nst_vec,
               B, R, C, D, FR, FC):
    info = plsc.get_sparse_core_info()
    NC, NS = info.num_cores, info.num_subcores
    NW = NC * NS
    P = 2 * B
    PPW = P // NW
    NCH = PPW // 128
    NG = PPW // 16

    mesh = plsc.VectorSubcoreMesh(core_axis_name="c", subcore_axis_name="s")

    @functools.partial(
        pl.kernel,
        mesh=mesh,
        out_type=[
            jax.ShapeDtypeStruct((P, D), jnp.float32),
            jax.ShapeDtypeStruct((P, D), jnp.float32),
            jax.ShapeDtypeStruct((P,), jnp.float32),
            jax.ShapeDtypeStruct((P,), jnp.float32),
        ],
        scratch_types=[
            pltpu.VMEM((PPW,), jnp.int32),
            pltpu.VMEM((PPW,), jnp.int32),
            pltpu.VMEM((PPW,), jnp.int32),
            pltpu.VMEM((PPW,), jnp.int32),
            pltpu.VMEM((PPW,), jnp.int32),
            pltpu.VMEM((16,), jnp.int32),
            pltpu.VMEM((16,), jnp.int32),
            pltpu.VMEM((NCH, 128), jnp.int32),
            pltpu.VMEM((NCH, 128), jnp.int32),
            pltpu.VMEM((NCH, 128), jnp.int32),
            pltpu.VMEM((NCH, 128), jnp.int32),
            pltpu.VMEM((PPW,), jnp.float32),
            pltpu.VMEM((PPW,), jnp.float32),
            pltpu.VMEM((PPW, D), jnp.float32),
            pltpu.VMEM((PPW, D), jnp.float32),
            pltpu.SemaphoreType.DMA,
            pltpu.SemaphoreType.DMA,
            pltpu.SemaphoreType.DMA,
            pltpu.SemaphoreType.DMA,
        ],
    )
    def sc_kernel(row_hbm, col_hbm, flow_hbm, idx_hbm, step_hbm,
                  boffr_hbm, bofff_hbm, boffc_hbm, nlt_hbm, nst_hbm,
                  lot_out, col_out, mlot_out, mcol_out,
                  idx_v, step_v, boffr_v, bofff_v, boffc_v, nlt_v, nst_v,
                  lotg_v, flowg_v, colg_v, flowval_v,
                  mlot_v, mcol_v, lotrows_v, colrows_v, sem0, sem1, sem2, sem3):
        wid = lax.axis_index("s") * NC + lax.axis_index("c")
        base = wid * PPW

        pltpu.sync_copy(idx_hbm.at[pl.ds(base, PPW)], idx_v)
        pltpu.sync_copy(step_hbm.at[pl.ds(base, PPW)], step_v)
        pltpu.sync_copy(boffr_hbm.at[pl.ds(base, PPW)], boffr_v)
        pltpu.sync_copy(bofff_hbm.at[pl.ds(base, PPW)], bofff_v)
        pltpu.sync_copy(boffc_hbm.at[pl.ds(base, PPW)], boffc_v)
        pltpu.sync_copy(nlt_hbm, nlt_v)
        pltpu.sync_copy(nst_hbm, nst_v)
        nlt = nlt_v[...]
        nst = nst_v[...]

        for g in range(NG):
            s = g * 16
            j, r = divmod(s, 128)
            idx = idx_v[pl.ds(s, 16)]
            stp = step_v[pl.ds(s, 16)]
            vlot = idx <= nlt
            safe_lot = jnp.minimum(jnp.where(vlot, idx, 0), R - 1)
            lotg_v[j, pl.ds(r, 16)] = boffr_v[pl.ds(s, 16)] + safe_lot
            mlot_v[pl.ds(s, 16)] = jnp.where(vlot, 1.0, 0.0).astype(jnp.float32)
            nxt = stp + 1
            dns = jnp.where(nxt > nst, 0, jnp.minimum(nxt, FC - 1))
            flowg_v[j, pl.ds(r, 16)] = (bofff_v[pl.ds(s, 16)]
                                        + safe_lot * FC + dns)

        fcps = [pltpu.async_copy(flow_hbm.at[flowg_v.at[j]], flowval_v.at[j],
                                 (sem0, sem1)[j % 2]) for j in range(NCH)]
        for cp in fcps:
            cp.wait()

        for g in range(NG):
            s = g * 16
            j, r = divmod(s, 128)
            ns = flowval_v[j, pl.ds(r, 16)]
            nxt = step_v[pl.ds(s, 16)] + 1
            vcol = jnp.logical_and(nxt <= nst,
                                   jnp.logical_and(ns >= 1, ns <= C))
            safe_col = jnp.where(vcol, ns - 1, 0)
            colg_v[j, pl.ds(r, 16)] = boffc_v[pl.ds(s, 16)] + safe_col
            mcol_v[pl.ds(s, 16)] = jnp.where(vcol, 1.0, 0.0).astype(jnp.float32)

        cps = []
        for j in range(NCH):
            cps.append(pltpu.async_copy(row_hbm.at[lotg_v.at[j]],
                                        lotrows_v.at[pl.ds(j * 128, 128)], sem0))
            cps.append(pltpu.async_copy(col_hbm.at[colg_v.at[j]],
                                        colrows_v.at[pl.ds(j * 128, 128)], sem1))
        for cp in cps:
            cp.wait()

        pltpu.sync_copy(lotrows_v, lot_out.at[pl.ds(base, PPW)])
        pltpu.sync_copy(colrows_v, col_out.at[pl.ds(base, PPW)])
        pltpu.sync_copy(mlot_v, mlot_out.at[pl.ds(base, PPW)])
        pltpu.sync_copy(mcol_v, mcol_out.at[pl.ds(base, PPW)])

    return sc_kernel(row_flat, col_flat, flow_flat, idx_am, step_am,
                     boff_row, boff_flow, boff_col, nlt_vec, nst_vec)


def _tc_combine(lot_rows, col_rows, m_lot, m_col, wt, B, D):
    BB = 256
    grid = (B // BB,)

    def body(lot_ref, col_ref, mlot_ref, mcol_ref, wt_ref, out_ref):
        e0 = mlot_ref[0] * lot_ref[0] + mcol_ref[0] * col_ref[0]
        e1 = mlot_ref[1] * lot_ref[1] + mcol_ref[1] * col_ref[1]
        out_ref[...] = (
            jnp.dot(e0, wt_ref[:D], preferred_element_type=jnp.float32)
            + jnp.dot(e1, wt_ref[D:], preferred_element_type=jnp.float32))

    return pl.pallas_call(
        body,
        grid=grid,
        in_specs=[
            pl.BlockSpec((2, BB, D), lambda i: (0, i, 0)),
            pl.BlockSpec((2, BB, D), lambda i: (0, i, 0)),
            pl.BlockSpec((2, BB, 1), lambda i: (0, i, 0)),
            pl.BlockSpec((2, BB, 1), lambda i: (0, i, 0)),
            pl.BlockSpec((2 * D, D), lambda i: (0, 0)),
        ],
        out_specs=pl.BlockSpec((BB, D), lambda i: (i, 0)),
        out_shape=jax.ShapeDtypeStruct((B, D), jnp.float32),
    )(lot_rows, col_rows, m_lot, m_col, wt)


def kernel(encoded_row, encoded_col, W, robot_lot_idx, robot_lot_step, flow,
           num_lot_type, num_step):
    B, R, D = encoded_row.shape
    C = encoded_col.shape[1]
    FR = flow.shape[1] * flow.shape[2]
    FC = flow.shape[2]

    row_flat = encoded_row.reshape(B * R, D)
    col_flat = encoded_col.reshape(B * C, D)
    flow_flat = flow.reshape(-1).astype(jnp.int32)
    idx_am = robot_lot_idx.T.reshape(-1).astype(jnp.int32)
    step_am = robot_lot_step.T.reshape(-1).astype(jnp.int32)
    nlt_vec = jnp.full((16,), num_lot_type, jnp.int32)
    nst_vec = jnp.full((16,), num_step, jnp.int32)
    bvec = jnp.tile(jnp.arange(B, dtype=jnp.int32), 2)
    boff_row = bvec * R
    boff_flow = bvec * FR
    boff_col = bvec * C

    lot_rows, col_rows, m_lot, m_col = _sc_gather(
        row_flat, col_flat, flow_flat, idx_am, step_am,
        boff_row, boff_flow, boff_col, nlt_vec, nst_vec,
        B, R, C, D, FR, FC)

    out = _tc_combine(
        lot_rows.reshape(2, B, D), col_rows.reshape(2, B, D),
        m_lot.reshape(2, B, 1), m_col.reshape(2, B, 1),
        W.T, B, D)
    return out


# speedup vs baseline: 5.7253x; 2.4146x over previous
import functools

import jax
import jax.numpy as jnp
from jax import lax
from jax.experimental import pallas as pl
from jax.experimental.pallas import tpu as pltpu
from jax.experimental.pallas import tpu_sc as plsc


def _sc_gather(row_flat, col_flat, idx_am, step_am, ns_am,
               boff_row, boff_col, nlt_vec, nst_vec, B, R, C, D):
    info = plsc.get_sparse_core_info()
    NC, NS = info.num_cores, info.num_subcores
    NW = NC * NS
    P = 2 * B
    PPW = P // NW
    NCH = PPW // 128
    NG = PPW // 16

    mesh = plsc.VectorSubcoreMesh(core_axis_name="c", subcore_axis_name="s")

    @functools.partial(
        pl.kernel,
        mesh=mesh,
        out_type=[
            jax.ShapeDtypeStruct((P, D), jnp.float32),
            jax.ShapeDtypeStruct((P, D), jnp.float32),
            jax.ShapeDtypeStruct((P,), jnp.float32),
            jax.ShapeDtypeStruct((P,), jnp.float32),
        ],
        scratch_types=[
            pltpu.VMEM((PPW,), jnp.int32),
            pltpu.VMEM((PPW,), jnp.int32),
            pltpu.VMEM((PPW,), jnp.int32),
            pltpu.VMEM((PPW,), jnp.int32),
            pltpu.VMEM((PPW,), jnp.int32),
            pltpu.VMEM((16,), jnp.int32),
            pltpu.VMEM((16,), jnp.int32),
            pltpu.VMEM((NCH, 128), jnp.int32),
            pltpu.VMEM((NCH, 128), jnp.int32),
            pltpu.VMEM((PPW,), jnp.float32),
            pltpu.VMEM((PPW,), jnp.float32),
            pltpu.VMEM((PPW, D), jnp.float32),
            pltpu.VMEM((PPW, D), jnp.float32),
            pltpu.SemaphoreType.DMA,
            pltpu.SemaphoreType.DMA,
        ],
    )
    def sc_kernel(row_hbm, col_hbm, idx_hbm, step_hbm, ns_hbm,
                  boffr_hbm, boffc_hbm, nlt_hbm, nst_hbm,
                  lot_out, col_out, mlot_out, mcol_out,
                  idx_v, step_v, ns_v, boffr_v, boffc_v, nlt_v, nst_v,
                  lotg_v, colg_v, mlot_v, mcol_v, lotrows_v, colrows_v,
                  sem0, sem1):
        wid = lax.axis_index("s") * NC + lax.axis_index("c")
        base = wid * PPW

        pltpu.sync_copy(idx_hbm.at[pl.ds(base, PPW)], idx_v)
        pltpu.sync_copy(step_hbm.at[pl.ds(base, PPW)], step_v)
        pltpu.sync_copy(ns_hbm.at[pl.ds(base, PPW)], ns_v)
        pltpu.sync_copy(boffr_hbm.at[pl.ds(base, PPW)], boffr_v)
        pltpu.sync_copy(boffc_hbm.at[pl.ds(base, PPW)], boffc_v)
        pltpu.sync_copy(nlt_hbm, nlt_v)
        pltpu.sync_copy(nst_hbm, nst_v)
        nlt = nlt_v[...]
        nst = nst_v[...]

        for g in range(NG):
            s = g * 16
            j, r = divmod(s, 128)
            idx = idx_v[pl.ds(s, 16)]
            stp = step_v[pl.ds(s, 16)]
            ns = ns_v[pl.ds(s, 16)]
            vlot = idx <= nlt
            safe_lot = jnp.minimum(jnp.where(vlot, idx, 0), R - 1)
            lotg_v[j, pl.ds(r, 16)] = boffr_v[pl.ds(s, 16)] + safe_lot
            mlot_v[pl.ds(s, 16)] = jnp.where(vlot, 1.0, 0.0).astype(jnp.float32)
            vcol = jnp.logical_and(stp + 1 <= nst,
                                   jnp.logical_and(ns >= 1, ns <= C))
            safe_col = jnp.where(vcol, ns - 1, 0)
            colg_v[j, pl.ds(r, 16)] = boffc_v[pl.ds(s, 16)] + safe_col
            mcol_v[pl.ds(s, 16)] = jnp.where(vcol, 1.0, 0.0).astype(jnp.float32)

        cps = []
        for j in range(NCH):
            cps.append(pltpu.async_copy(row_hbm.at[lotg_v.at[j]],
                                        lotrows_v.at[pl.ds(j * 128, 128)], sem0))
            cps.append(pltpu.async_copy(col_hbm.at[colg_v.at[j]],
                                        colrows_v.at[pl.ds(j * 128, 128)], sem1))
        for cp in cps:
            cp.wait()

        pltpu.sync_copy(lotrows_v, lot_out.at[pl.ds(base, PPW)])
        pltpu.sync_copy(colrows_v, col_out.at[pl.ds(base, PPW)])
        pltpu.sync_copy(mlot_v, mlot_out.at[pl.ds(base, PPW)])
        pltpu.sync_copy(mcol_v, mcol_out.at[pl.ds(base, PPW)])

    return sc_kernel(row_flat, col_flat, idx_am, step_am, ns_am,
                     boff_row, boff_col, nlt_vec, nst_vec)


def _tc_combine(lot_rows, col_rows, m_lot, m_col, wt, B, D):
    BB = 256
    grid = (B // BB,)

    def body(lot_ref, col_ref, mlot_ref, mcol_ref, wt_ref, out_ref):
        e0 = mlot_ref[0] * lot_ref[0] + mcol_ref[0] * col_ref[0]
        e1 = mlot_ref[1] * lot_ref[1] + mcol_ref[1] * col_ref[1]
        out_ref[...] = (
            jnp.dot(e0, wt_ref[:D], preferred_element_type=jnp.float32)
            + jnp.dot(e1, wt_ref[D:], preferred_element_type=jnp.float32))

    return pl.pallas_call(
        body,
        grid=grid,
        in_specs=[
            pl.BlockSpec((2, BB, D), lambda i: (0, i, 0)),
            pl.BlockSpec((2, BB, D), lambda i: (0, i, 0)),
            pl.BlockSpec((2, BB, 1), lambda i: (0, i, 0)),
            pl.BlockSpec((2, BB, 1), lambda i: (0, i, 0)),
            pl.BlockSpec((2 * D, D), lambda i: (0, 0)),
        ],
        out_specs=pl.BlockSpec((BB, D), lambda i: (i, 0)),
        out_shape=jax.ShapeDtypeStruct((B, D), jnp.float32),
    )(lot_rows, col_rows, m_lot, m_col, wt)


def kernel(encoded_row, encoded_col, W, robot_lot_idx, robot_lot_step, flow,
           num_lot_type, num_step):
    B, R, D = encoded_row.shape
    C = encoded_col.shape[1]

    row_flat = encoded_row.reshape(B * R, D)
    col_flat = encoded_col.reshape(B * C, D)
    idx_am = robot_lot_idx.T.reshape(-1).astype(jnp.int32)
    step_am = robot_lot_step.T.reshape(-1).astype(jnp.int32)
    nlt_vec = jnp.full((16,), num_lot_type, jnp.int32)
    nst_vec = jnp.full((16,), num_step, jnp.int32)
    bvec = jnp.tile(jnp.arange(B, dtype=jnp.int32), 2)
    boff_row = bvec * R
    boff_col = bvec * C

    next_step = robot_lot_step + 1
    dns = jnp.where(next_step > num_step, 0, next_step)
    lot_f = jnp.where(robot_lot_idx <= num_lot_type, robot_lot_idx, 0)
    ns = flow[jnp.arange(B)[:, None], lot_f, dns]
    ns_am = ns.T.reshape(-1).astype(jnp.int32)

    lot_rows, col_rows, m_lot, m_col = _sc_gather(
        row_flat, col_flat, idx_am, step_am, ns_am,
        boff_row, boff_col, nlt_vec, nst_vec, B, R, C, D)

    out = _tc_combine(
        lot_rows.reshape(2, B, D), col_rows.reshape(2, B, D),
        m_lot.reshape(2, B, 1), m_col.reshape(2, B, 1),
        W.T, B, D)
    return out


# speedup vs baseline: 7.7236x; 1.3490x over previous
import functools

import jax
import jax.numpy as jnp
from jax import lax
from jax.experimental import pallas as pl
from jax.experimental.pallas import tpu as pltpu
from jax.experimental.pallas import tpu_sc as plsc


def _sc_gather(row_flat, col_flat, aux, NW, PPW, AUXW, R, C, D, two_boffs):
    P = NW * PPW
    NCH = PPW // 128
    NG = PPW // 16
    NSEG = 5 if two_boffs else 4
    DG = D // 16

    mesh = plsc.VectorSubcoreMesh(core_axis_name="c", subcore_axis_name="s")

    @functools.partial(
        pl.kernel,
        mesh=mesh,
        out_type=[
            jax.ShapeDtypeStruct((P, D), jnp.float32),
            jax.ShapeDtypeStruct((P, D), jnp.float32),
        ],
        scratch_types=[
            pltpu.VMEM((AUXW,), jnp.int32),
            pltpu.VMEM((NCH, 128), jnp.int32),
            pltpu.VMEM((NCH, 128), jnp.int32),
            pltpu.VMEM((PPW,), jnp.float32),
            pltpu.VMEM((PPW,), jnp.float32),
            pltpu.VMEM((PPW, D), jnp.float32),
            pltpu.VMEM((PPW, D), jnp.float32),
            pltpu.SemaphoreType.DMA,
            pltpu.SemaphoreType.DMA,
            pltpu.SemaphoreType.DMA,
            pltpu.SemaphoreType.DMA,
        ],
    )
    def sc_kernel(row_hbm, col_hbm, aux_hbm, lot_out, col_out,
                  aux_v, lotg_v, colg_v, mlot_v, mcol_v,
                  lotrows_v, colrows_v, sem0, sem1, sem2, sem3):
        wid = lax.axis_index("s") * 2 + lax.axis_index("c")
        base = wid * PPW

        pltpu.sync_copy(aux_hbm.at[pl.ds(wid * AUXW, AUXW)], aux_v)
        nlt = aux_v[pl.ds(NSEG * PPW, 16)]
        nst = aux_v[pl.ds(NSEG * PPW + 16, 16)]

        for g in range(NG):
            s = g * 16
            j, r = divmod(s, 128)
            idx = aux_v[pl.ds(s, 16)]
            stp = aux_v[pl.ds(PPW + s, 16)]
            ns = aux_v[pl.ds(2 * PPW + s, 16)]
            boffr = aux_v[pl.ds(3 * PPW + s, 16)]
            boffc = aux_v[pl.ds(4 * PPW + s, 16)] if two_boffs else boffr
            vlot = idx <= nlt
            safe_lot = jnp.minimum(jnp.where(vlot, idx, 0), R - 1)
            lotg_v[j, pl.ds(r, 16)] = boffr + safe_lot
            mlot_v[pl.ds(s, 16)] = jnp.where(vlot, 1.0, 0.0)
            vcol = jnp.logical_and(stp + 1 <= nst,
                                   jnp.logical_and(ns >= 1, ns <= C))
            safe_col = jnp.where(vcol, ns - 1, 0)
            colg_v[j, pl.ds(r, 16)] = boffc + safe_col
            mcol_v[pl.ds(s, 16)] = jnp.where(vcol, 1.0, 0.0)

        lot_cps, col_cps = [], []
        for j in range(NCH):
            lot_cps.append(pltpu.async_copy(
                row_hbm.at[lotg_v.at[j]],
                lotrows_v.at[pl.ds(j * 128, 128)], sem0))
            col_cps.append(pltpu.async_copy(
                col_hbm.at[colg_v.at[j]],
                colrows_v.at[pl.ds(j * 128, 128)], sem1))
        for cp in lot_cps:
            cp.wait()

        zeros16 = jnp.zeros((16,), jnp.float32)

        def make_zero_scan(mask_v, rows_v):
            def zero_scan(g, carry):
                s16 = g * 16
                m16 = mask_v[pl.ds(s16, 16)]
                for l in range(16):
                    @pl.when(m16[l] == 0.0)
                    def _():
                        for gg in range(DG):
                            rows_v[s16 + l, pl.ds(gg * 16, 16)] = zeros16
                return carry
            return zero_scan

        lax.fori_loop(0, NG, make_zero_scan(mlot_v, lotrows_v), 0)
        lot_wb = pltpu.async_copy(lotrows_v, lot_out.at[pl.ds(base, PPW)], sem2)

        for cp in col_cps:
            cp.wait()
        lax.fori_loop(0, NG, make_zero_scan(mcol_v, colrows_v), 0)
        col_wb = pltpu.async_copy(colrows_v, col_out.at[pl.ds(base, PPW)], sem3)
        lot_wb.wait()
        col_wb.wait()

    return sc_kernel(row_flat, col_flat, aux)


def _tc_combine(lot_rows, col_rows, W, B, D):
    BB = 256
    grid = (B // BB,)
    dn = (((1,), (1,)), ((), ()))

    def body(lot_ref, col_ref, w_ref, out_ref):
        e0 = lot_ref[0] + col_ref[0]
        e1 = lot_ref[1] + col_ref[1]
        out_ref[...] = (
            lax.dot_general(e0, w_ref[:, :D], dn,
                            preferred_element_type=jnp.float32)
            + lax.dot_general(e1, w_ref[:, D:], dn,
                              preferred_element_type=jnp.float32))

    return pl.pallas_call(
        body,
        grid=grid,
        in_specs=[
            pl.BlockSpec((2, BB, D), lambda i: (0, i, 0)),
            pl.BlockSpec((2, BB, D), lambda i: (0, i, 0)),
            pl.BlockSpec((D, 2 * D), lambda i: (0, 0)),
        ],
        out_specs=pl.BlockSpec((BB, D), lambda i: (i, 0)),
        out_shape=jax.ShapeDtypeStruct((B, D), jnp.float32),
    )(lot_rows, col_rows, W)


def kernel(encoded_row, encoded_col, W, robot_lot_idx, robot_lot_step, flow,
           num_lot_type, num_step):
    B, R, D = encoded_row.shape
    C = encoded_col.shape[1]

    row_flat = encoded_row.reshape(B * R, D)
    col_flat = encoded_col.reshape(B * C, D)
    idx_am = robot_lot_idx.T.reshape(-1).astype(jnp.int32)
    step_am = robot_lot_step.T.reshape(-1).astype(jnp.int32)

    next_step = robot_lot_step + 1
    dns = jnp.where(next_step > num_step, 0, next_step)
    lot_f = jnp.where(robot_lot_idx <= num_lot_type, robot_lot_idx, 0)
    ns = flow[jnp.arange(B)[:, None], lot_f, dns]
    ns_am = ns.T.reshape(-1).astype(jnp.int32)

    info = plsc.get_sparse_core_info()
    NW = info.num_cores * info.num_subcores
    P = 2 * B
    PPW = P // NW
    bvec = jnp.tile(jnp.arange(B, dtype=jnp.int32), 2)
    segs = [idx_am, step_am, ns_am, bvec * R]
    if R != C:
        segs.append(bvec * C)
    AUXW = len(segs) * PPW + 32
    aux = jnp.concatenate(
        [jnp.stack([s.reshape(NW, PPW) for s in segs], axis=1).reshape(
            NW, len(segs) * PPW),
         jnp.full((NW, 16), num_lot_type, jnp.int32),
         jnp.full((NW, 16), num_step, jnp.int32)], axis=1).reshape(-1)

    lot_rows, col_rows = _sc_gather(
        row_flat, col_flat, aux, NW, PPW, AUXW, R, C, D, R != C)

    return _tc_combine(lot_rows.reshape(2, B, D), col_rows.reshape(2, B, D),
                       W, B, D)


# speedup vs baseline: 8.8133x; 1.1411x over previous
import functools

import jax
import jax.numpy as jnp
from jax import lax
from jax.experimental import pallas as pl
from jax.experimental.pallas import tpu as pltpu
from jax.experimental.pallas import tpu_sc as plsc


def _sc_gather(row_flat, col_flat, aux, ns_am, NW, PPW, AUXW, R, C, D,
               two_boffs):
    P = NW * PPW
    NCH = PPW // 128
    GPC = 128 // 16
    NSEG = 4 if two_boffs else 3
    DG = D // 16

    mesh = plsc.VectorSubcoreMesh(core_axis_name="c", subcore_axis_name="s")

    @functools.partial(
        pl.kernel,
        mesh=mesh,
        out_type=[
            jax.ShapeDtypeStruct((P, D), jnp.float32),
            jax.ShapeDtypeStruct((P, D), jnp.float32),
        ],
        scratch_types=[
            pltpu.VMEM((AUXW,), jnp.int32),
            pltpu.VMEM((PPW,), jnp.int32),
            pltpu.VMEM((NCH, 128), jnp.int32),
            pltpu.VMEM((NCH, 128), jnp.int32),
            pltpu.VMEM((PPW,), jnp.float32),
            pltpu.VMEM((PPW,), jnp.float32),
            pltpu.VMEM((PPW, D), jnp.float32),
            pltpu.VMEM((PPW, D), jnp.float32),
            pltpu.SemaphoreType.DMA,
            pltpu.SemaphoreType.DMA,
            pltpu.SemaphoreType.DMA,
            pltpu.SemaphoreType.DMA,
        ],
    )
    def sc_kernel(row_hbm, col_hbm, aux_hbm, ns_hbm, lot_out, col_out,
                  aux_v, ns_v, lotg_v, colg_v, mlot_v, mcol_v,
                  lotrows_v, colrows_v, sem0, sem1, sem2, sem3):
        wid = lax.axis_index("s") * 2 + lax.axis_index("c")
        base = wid * PPW

        pltpu.sync_copy(aux_hbm.at[pl.ds(wid * AUXW, AUXW)], aux_v)
        pltpu.sync_copy(ns_hbm.at[pl.ds(base, PPW)], ns_v)
        nlt = aux_v[pl.ds(NSEG * PPW, 16)]
        nst = aux_v[pl.ds(NSEG * PPW + 16, 16)]

        lot_cps, col_cps = [], []
        for j in range(NCH):
            for gc in range(GPC):
                s = j * 128 + gc * 16
                r = gc * 16
                idx = aux_v[pl.ds(s, 16)]
                stp = aux_v[pl.ds(PPW + s, 16)]
                ns = ns_v[pl.ds(s, 16)]
                boffr = aux_v[pl.ds(2 * PPW + s, 16)]
                boffc = aux_v[pl.ds(3 * PPW + s, 16)] if two_boffs else boffr
                vlot = idx <= nlt
                safe_lot = jnp.minimum(jnp.where(vlot, idx, 0), R - 1)
                lotg_v[j, pl.ds(r, 16)] = boffr + safe_lot
                mlot_v[pl.ds(s, 16)] = jnp.where(vlot, 1.0, 0.0)
                vcol = jnp.logical_and(stp + 1 <= nst,
                                       jnp.logical_and(ns >= 1, ns <= C))
                safe_col = jnp.where(vcol, ns - 1, 0)
                colg_v[j, pl.ds(r, 16)] = boffc + safe_col
                mcol_v[pl.ds(s, 16)] = jnp.where(vcol, 1.0, 0.0)
            lot_cps.append(pltpu.async_copy(
                row_hbm.at[lotg_v.at[j]],
                lotrows_v.at[pl.ds(j * 128, 128)], sem0))
            col_cps.append(pltpu.async_copy(
                col_hbm.at[colg_v.at[j]],
                colrows_v.at[pl.ds(j * 128, 128)], sem1))

        zeros16 = jnp.zeros((16,), jnp.float32)

        def make_zero_scan(mask_v, rows_v):
            def zero_scan(g, carry):
                s16 = g * 16
                m16 = mask_v[pl.ds(s16, 16)]
                for l in range(16):
                    @pl.when(m16[l] == 0.0)
                    def _():
                        for gg in range(DG):
                            rows_v[s16 + l, pl.ds(gg * 16, 16)] = zeros16
                return carry
            return zero_scan

        for cp in lot_cps:
            cp.wait()
        lax.fori_loop(0, PPW // 16, make_zero_scan(mlot_v, lotrows_v), 0)
        lot_wb = pltpu.async_copy(lotrows_v, lot_out.at[pl.ds(base, PPW)], sem2)

        for cp in col_cps:
            cp.wait()
        lax.fori_loop(0, PPW // 16, make_zero_scan(mcol_v, colrows_v), 0)
        col_wb = pltpu.async_copy(colrows_v, col_out.at[pl.ds(base, PPW)], sem3)
        lot_wb.wait()
        col_wb.wait()

    return sc_kernel(row_flat, col_flat, aux, ns_am)


def _tc_combine(lot_rows, col_rows, W, B, D):
    BB = 512
    grid = (B // BB,)
    dn = (((1,), (1,)), ((), ()))

    def body(lot_ref, col_ref, w_ref, out_ref):
        e0 = lot_ref[0] + col_ref[0]
        e1 = lot_ref[1] + col_ref[1]
        out_ref[...] = (
            lax.dot_general(e0, w_ref[:, :D], dn,
                            preferred_element_type=jnp.float32)
            + lax.dot_general(e1, w_ref[:, D:], dn,
                              preferred_element_type=jnp.float32))

    return pl.pallas_call(
        body,
        grid=grid,
        in_specs=[
            pl.BlockSpec((2, BB, D), lambda i: (0, i, 0)),
            pl.BlockSpec((2, BB, D), lambda i: (0, i, 0)),
            pl.BlockSpec((D, 2 * D), lambda i: (0, 0)),
        ],
        out_specs=pl.BlockSpec((BB, D), lambda i: (i, 0)),
        out_shape=jax.ShapeDtypeStruct((B, D), jnp.float32),
    )(lot_rows, col_rows, W)


def kernel(encoded_row, encoded_col, W, robot_lot_idx, robot_lot_step, flow,
           num_lot_type, num_step):
    B, R, D = encoded_row.shape
    C = encoded_col.shape[1]

    row_flat = encoded_row.reshape(B * R, D)
    col_flat = encoded_col.reshape(B * C, D)
    idx_am = robot_lot_idx.T.reshape(-1).astype(jnp.int32)
    step_am = robot_lot_step.T.reshape(-1).astype(jnp.int32)

    b_am = jnp.tile(jnp.arange(B, dtype=jnp.int32), 2)
    dns_am = jnp.where(step_am + 1 > num_step, 0, step_am + 1)
    lot_f_am = jnp.where(idx_am <= num_lot_type, idx_am, 0)
    ns_am = flow[b_am, lot_f_am, dns_am].astype(jnp.int32)

    info = plsc.get_sparse_core_info()
    NW = info.num_cores * info.num_subcores
    P = 2 * B
    PPW = P // NW
    segs = [idx_am, step_am, b_am * R]
    if R != C:
        segs.append(b_am * C)
    AUXW = len(segs) * PPW + 32
    aux = jnp.concatenate(
        [jnp.stack([s.reshape(NW, PPW) for s in segs], axis=1).reshape(
            NW, len(segs) * PPW),
         jnp.full((NW, 16), num_lot_type, jnp.int32),
         jnp.full((NW, 16), num_step, jnp.int32)], axis=1).reshape(-1)

    lot_rows, col_rows = _sc_gather(
        row_flat, col_flat, aux, ns_am, NW, PPW, AUXW, R, C, D, R != C)

    return _tc_combine(lot_rows.reshape(2, B, D), col_rows.reshape(2, B, D),
                       W, B, D)


# speedup vs baseline: 9.0433x; 1.0261x over previous
import functools

import jax
import jax.numpy as jnp
from jax import lax
from jax.experimental import pallas as pl
from jax.experimental.pallas import tpu as pltpu
from jax.experimental.pallas import tpu_sc as plsc


def _sc_gather(row_flat, col_flat, aux, ns_am, NW, PPW, AUXW, R, C, D,
               two_boffs):
    P = NW * PPW
    NCH = PPW // 128
    GPC = 128 // 16
    NSEG = 4 if two_boffs else 3
    DG = D // 16

    mesh = plsc.VectorSubcoreMesh(core_axis_name="c", subcore_axis_name="s")

    @functools.partial(
        pl.kernel,
        mesh=mesh,
        out_type=[
            jax.ShapeDtypeStruct((P, D), jnp.float32),
            jax.ShapeDtypeStruct((P, D), jnp.float32),
        ],
        scratch_types=[
            pltpu.VMEM((AUXW,), jnp.int32),
            pltpu.VMEM((PPW,), jnp.int32),
            pltpu.VMEM((NCH, 128), jnp.int32),
            pltpu.VMEM((NCH, 128), jnp.int32),
            pltpu.VMEM((PPW,), jnp.float32),
            pltpu.VMEM((PPW,), jnp.float32),
            pltpu.VMEM((PPW, D), jnp.float32),
            pltpu.VMEM((PPW, D), jnp.float32),
            pltpu.SemaphoreType.DMA,
            pltpu.SemaphoreType.DMA,
            pltpu.SemaphoreType.DMA,
            pltpu.SemaphoreType.DMA,
        ],
    )
    def sc_kernel(row_hbm, col_hbm, aux_hbm, ns_hbm, lot_out, col_out,
                  aux_v, ns_v, lotg_v, colg_v, mlot_v, mcol_v,
                  lotrows_v, colrows_v, sem0, sem1, sem2, sem3):
        wid = lax.axis_index("s") * 2 + lax.axis_index("c")
        base = wid * PPW

        aux_cp = pltpu.async_copy(aux_hbm.at[pl.ds(wid * AUXW, AUXW)], aux_v,
                                  sem2)
        ns_cp = pltpu.async_copy(ns_hbm.at[pl.ds(base, PPW)], ns_v, sem3)
        aux_cp.wait()
        ns_cp.wait()
        nlt = aux_v[pl.ds(NSEG * PPW, 16)]
        nst = aux_v[pl.ds(NSEG * PPW + 16, 16)]

        lot_cps, col_cps = [], []
        for j in range(NCH):
            for gc in range(GPC):
                s = j * 128 + gc * 16
                r = gc * 16
                idx = aux_v[pl.ds(s, 16)]
                stp = aux_v[pl.ds(PPW + s, 16)]
                ns = ns_v[pl.ds(s, 16)]
                boffr = aux_v[pl.ds(2 * PPW + s, 16)]
                boffc = aux_v[pl.ds(3 * PPW + s, 16)] if two_boffs else boffr
                vlot = idx <= nlt
                safe_lot = jnp.minimum(jnp.where(vlot, idx, 0), R - 1)
                lotg_v[j, pl.ds(r, 16)] = boffr + safe_lot
                mlot_v[pl.ds(s, 16)] = jnp.where(vlot, 1.0, 0.0)
                vcol = jnp.logical_and(stp + 1 <= nst,
                                       jnp.logical_and(ns >= 1, ns <= C))
                safe_col = jnp.where(vcol, ns - 1, 0)
                colg_v[j, pl.ds(r, 16)] = boffc + safe_col
                mcol_v[pl.ds(s, 16)] = jnp.where(vcol, 1.0, 0.0)
            lot_cps.append(pltpu.async_copy(
                row_hbm.at[lotg_v.at[j]],
                lotrows_v.at[pl.ds(j * 128, 128)], sem0))
            col_cps.append(pltpu.async_copy(
                col_hbm.at[colg_v.at[j]],
                colrows_v.at[pl.ds(j * 128, 128)], sem1))

        zeros16 = jnp.zeros((16,), jnp.float32)

        def make_zero_scan(mask_v, rows_v):
            def zero_scan(g, carry):
                s16 = g * 16
                m16 = mask_v[pl.ds(s16, 16)]
                for l in range(16):
                    @pl.when(m16[l] == 0.0)
                    def _():
                        for gg in range(DG):
                            rows_v[s16 + l, pl.ds(gg * 16, 16)] = zeros16
                return carry
            return zero_scan

        wbs = []
        for j in range(NCH):
            lot_cps[j].wait()
            lax.fori_loop(j * GPC, (j + 1) * GPC,
                          make_zero_scan(mlot_v, lotrows_v), 0)
            wbs.append(pltpu.async_copy(
                lotrows_v.at[pl.ds(j * 128, 128)],
                lot_out.at[pl.ds(base + j * 128, 128)], sem2))
            col_cps[j].wait()
            lax.fori_loop(j * GPC, (j + 1) * GPC,
                          make_zero_scan(mcol_v, colrows_v), 0)
            wbs.append(pltpu.async_copy(
                colrows_v.at[pl.ds(j * 128, 128)],
                col_out.at[pl.ds(base + j * 128, 128)], sem3))
        for wb in wbs:
            wb.wait()

    return sc_kernel(row_flat, col_flat, aux, ns_am)


def _tc_combine(lot_rows, col_rows, W, B, D):
    BB = 512
    grid = (B // BB,)
    dn = (((1,), (1,)), ((), ()))

    def body(lot_ref, col_ref, w_ref, out_ref):
        e0 = lot_ref[0] + col_ref[0]
        e1 = lot_ref[1] + col_ref[1]
        out_ref[...] = (
            lax.dot_general(e0, w_ref[:, :D], dn,
                            preferred_element_type=jnp.float32)
            + lax.dot_general(e1, w_ref[:, D:], dn,
                              preferred_element_type=jnp.float32))

    return pl.pallas_call(
        body,
        grid=grid,
        in_specs=[
            pl.BlockSpec((2, BB, D), lambda i: (0, i, 0)),
            pl.BlockSpec((2, BB, D), lambda i: (0, i, 0)),
            pl.BlockSpec((D, 2 * D), lambda i: (0, 0)),
        ],
        out_specs=pl.BlockSpec((BB, D), lambda i: (i, 0)),
        out_shape=jax.ShapeDtypeStruct((B, D), jnp.float32),
    )(lot_rows, col_rows, W)


def kernel(encoded_row, encoded_col, W, robot_lot_idx, robot_lot_step, flow,
           num_lot_type, num_step):
    B, R, D = encoded_row.shape
    C = encoded_col.shape[1]

    row_flat = encoded_row.reshape(B * R, D)
    col_flat = encoded_col.reshape(B * C, D)
    idx_am = robot_lot_idx.T.reshape(-1).astype(jnp.int32)
    step_am = robot_lot_step.T.reshape(-1).astype(jnp.int32)

    b_am = jnp.tile(jnp.arange(B, dtype=jnp.int32), 2)
    dns_am = jnp.where(step_am + 1 > num_step, 0, step_am + 1)
    lot_f_am = jnp.where(idx_am <= num_lot_type, idx_am, 0)
    ns_am = flow[b_am, lot_f_am, dns_am].astype(jnp.int32)

    info = plsc.get_sparse_core_info()
    NW = info.num_cores * info.num_subcores
    P = 2 * B
    PPW = P // NW
    segs = [idx_am, step_am, b_am * R]
    if R != C:
        segs.append(b_am * C)
    AUXW = len(segs) * PPW + 32
    aux = jnp.concatenate(
        [jnp.stack([s.reshape(NW, PPW) for s in segs], axis=1).reshape(
            NW, len(segs) * PPW),
         jnp.full((NW, 16), num_lot_type, jnp.int32),
         jnp.full((NW, 16), num_step, jnp.int32)], axis=1).reshape(-1)

    lot_rows, col_rows = _sc_gather(
        row_flat, col_flat, aux, ns_am, NW, PPW, AUXW, R, C, D, R != C)

    return _tc_combine(lot_rows.reshape(2, B, D), col_rows.reshape(2, B, D),
                       W, B, D)
